# baseline (speedup 1.0000x reference)
import functools

import jax
import jax.numpy as jnp
from jax import lax
from jax.experimental import pallas as pl
from jax.experimental.pallas import tpu as pltpu
from jax.experimental.pallas import tpu_sc as plsc

EPS = 1e-6
NC = 2
NS = 16
NW = NC * NS
L = 16
K = 128


def _sc_mesh():
    return plsc.VectorSubcoreMesh(
        core_axis_name="c", subcore_axis_name="s", num_cores=NC, num_subcores=NS
    )


def _make_degrees(E, N):
    assert E % NW == 0 and N % L == 0
    epw = E // NW
    assert epw % L == 0 and (epw * 4) % 8 == 0

    @functools.partial(
        pl.kernel,
        out_type=jax.ShapeDtypeStruct((NW * 2 * N,), jnp.float32),
        mesh=_sc_mesh(),
        compiler_params=pltpu.CompilerParams(needs_layout_passes=False),
        scratch_types=[
            pltpu.VMEM((N,), jnp.float32),
            pltpu.VMEM((N,), jnp.float32),
            pltpu.VMEM((epw,), jnp.int32),
            pltpu.VMEM((epw,), jnp.int32),
        ],
    )
    def deg_kernel(src_hbm, dst_hbm, out_hbm, hs, hd, si, di):
        c = lax.axis_index("c")
        s = lax.axis_index("s")
        wid = c * NS + s
        base = wid * epw
        zeros16 = jnp.zeros((L,), jnp.float32)
        ones16 = jnp.ones((L,), jnp.float32)

        pltpu.sync_copy(src_hbm.at[pl.ds(base, epw)], si)
        pltpu.sync_copy(dst_hbm.at[pl.ds(base, epw)], di)

        def zero_body(i, carry):
            hs[pl.ds(i * L, L)] = zeros16
            hd[pl.ds(i * L, L)] = zeros16
            return carry

        lax.fori_loop(0, N // L, zero_body, 0)

        def hist_body(i, carry):
            plsc.addupdate_scatter(hs, [si[pl.ds(i * L, L)]], ones16)
            plsc.addupdate_scatter(hd, [di[pl.ds(i * L, L)]], ones16)
            return carry

        lax.fori_loop(0, epw // L, hist_body, 0)

        pltpu.sync_copy(hs, out_hbm.at[pl.ds(wid * 2 * N, N)])
        pltpu.sync_copy(hd, out_hbm.at[pl.ds(wid * 2 * N + N, N)])

    return deg_kernel


def _make_aggregate(E_pad, N, D):
    assert E_pad % (NW * K) == 0
    cpt = E_pad // (NW * K)
    assert cpt % 8 == 0 and cpt >= 4
    NA = N + 8
    rpt = NA // NS // 8 * 8
    ztail = NA - NS * rpt
    etail = N - NS * rpt

    @functools.partial(
        pl.kernel,
        out_type=jax.ShapeDtypeStruct((NC, N, D), jnp.float32),
        mesh=_sc_mesh(),
        compiler_params=pltpu.CompilerParams(needs_layout_passes=False),
        scratch_types=[
            pltpu.VMEM_SHARED((NA, D), jnp.float32),
            pltpu.VMEM((cpt, K), jnp.int32),
            pltpu.VMEM((K,), jnp.int32),
            pltpu.VMEM((K,), jnp.int32),
            pltpu.VMEM((K, D), jnp.float32),
            pltpu.VMEM((K, D), jnp.float32),
            pltpu.SemaphoreType.DMA,
            pltpu.SemaphoreType.DMA,
            pltpu.SemaphoreType.DMA,
            pltpu.SemaphoreType.DMA,
        ],
    )
    def agg_kernel(h_hbm, src_hbm, dst2_hbm, zeros_hbm, out_hbm,
                   acc, di, ia, ib, rows_a, rows_b,
                   sem_a, sem_b, sem_ia, sem_ib):
        c = lax.axis_index("c")
        s = lax.axis_index("s")
        wid = c * NS + s
        ebase = wid * cpt * K

        pltpu.sync_copy(zeros_hbm.at[pl.ds(s * rpt, rpt)],
                        acc.at[pl.ds(s * rpt, rpt)])
        if ztail:
            @pl.when(s == NS - 1)
            def _():
                pltpu.sync_copy(zeros_hbm.at[pl.ds(NS * rpt, ztail)],
                                acc.at[pl.ds(NS * rpt, ztail)])

        pltpu.sync_copy(dst2_hbm.at[pl.ds(wid * cpt, cpt)], di)
        plsc.subcore_barrier()

        def idx_copy(j, buf, sem):
            pltpu.async_copy(src_hbm.at[pl.ds(ebase + j * K, K)], buf, sem)

        def idx_wait(buf, sem):
            pltpu.make_async_copy(src_hbm.at[pl.ds(ebase, K)], buf, sem).wait()

        def gather(buf_idx, buf, sem):
            pltpu.async_copy(h_hbm.at[buf_idx], buf, sem)

        def gather_wait(buf_idx, buf, sem):
            pltpu.make_async_copy(h_hbm.at[buf_idx], buf, sem).wait()

        def scatter(j, buf):
            pltpu.sync_copy(buf, acc.at[di.at[j]], add=True)

        idx_copy(0, ia, sem_ia)
        idx_wait(ia, sem_ia)
        gather(ia, rows_a, sem_a)
        idx_copy(1, ib, sem_ib)

        def pipe_body(i, carry):
            idx_wait(ib, sem_ib)
            gather(ib, rows_b, sem_b)
            gather_wait(ia, rows_a, sem_a)
            idx_copy(2 * i + 2, ia, sem_ia)
            scatter(2 * i, rows_a)
            idx_wait(ia, sem_ia)
            gather(ia, rows_a, sem_a)
            gather_wait(ib, rows_b, sem_b)
            idx_copy(2 * i + 3, ib, sem_ib)
            scatter(2 * i + 1, rows_b)
            return carry

        lax.fori_loop(0, cpt // 2 - 1, pipe_body, 0)

        idx_wait(ib, sem_ib)
        gather(ib, rows_b, sem_b)
        gather_wait(ia, rows_a, sem_a)
        scatter(cpt - 2, rows_a)
        gather_wait(ib, rows_b, sem_b)
        scatter(cpt - 1, rows_b)

        plsc.subcore_barrier()
        pltpu.sync_copy(acc.at[pl.ds(s * rpt, rpt)],
                        out_hbm.at[c, pl.ds(s * rpt, rpt)])
        if etail:
            @pl.when(s == NS - 1)
            def _():
                pltpu.sync_copy(acc.at[pl.ds(NS * rpt, etail)],
                                out_hbm.at[c, pl.ds(NS * rpt, etail)])

    return agg_kernel


def _prep(x, hist_t, a2, b2, block_n):
    N, D = x.shape

    def body(x_ref, hist_ref, a2_ref, b2_ref, h_ref):
        xb = x_ref[...]
        mean = jnp.mean(xb, axis=1, keepdims=True)
        xc = xb - mean
        std = jnp.sqrt(jnp.sum(xc * xc, axis=1, keepdims=True) / (D - 1))
        hn = a2_ref[...] * xc / (std + EPS) + b2_ref[...]
        out_deg = jnp.maximum(jnp.sum(hist_ref[...][0], axis=1), 1.0)
        h_ref[...] = hn * lax.rsqrt(out_deg)[:, None]

    return pl.pallas_call(
        body,
        grid=(N // block_n,),
        in_specs=[
            pl.BlockSpec((block_n, D), lambda i: (i, 0)),
            pl.BlockSpec((2, block_n, NW), lambda i: (0, i, 0)),
            pl.BlockSpec((1, D), lambda i: (0, 0)),
            pl.BlockSpec((1, D), lambda i: (0, 0)),
        ],
        out_specs=pl.BlockSpec((block_n, D), lambda i: (i, 0)),
        out_shape=jax.ShapeDtypeStruct((N, D), jnp.float32),
    )(x, hist_t, a2.reshape(1, D), b2.reshape(1, D))


def _finish(parts, hist_t, x, W, b, block_n):
    N, D = x.shape

    def body(parts_ref, hist_ref, x_ref, w_ref, b_ref, out_ref):
        agg = parts_ref[0] + parts_ref[1]
        in_deg = jnp.maximum(jnp.sum(hist_ref[...][1], axis=1), 1.0)
        agg = agg * lax.rsqrt(in_deg)[:, None]
        out = jnp.dot(agg, w_ref[...], preferred_element_type=jnp.float32)
        out_ref[...] = jnp.maximum(out + b_ref[...], 0.0) + x_ref[...]

    return pl.pallas_call(
        body,
        grid=(N // block_n,),
        in_specs=[
            pl.BlockSpec((NC, block_n, D), lambda i: (0, i, 0)),
            pl.BlockSpec((2, block_n, NW), lambda i: (0, i, 0)),
            pl.BlockSpec((block_n, D), lambda i: (i, 0)),
            pl.BlockSpec((D, D), lambda i: (0, 0)),
            pl.BlockSpec((1, D), lambda i: (0, 0)),
        ],
        out_specs=pl.BlockSpec((block_n, D), lambda i: (i, 0)),
        out_shape=jax.ShapeDtypeStruct((N, D), jnp.float32),
    )(parts, hist_t, x, W, b.reshape(1, D))


def kernel(x, edge_index, W, b, a2, b2):
    N, D = x.shape
    E = edge_index.shape[1]
    src = edge_index[0]
    dst = edge_index[1]

    hist = _make_degrees(E, N)(src, dst).reshape(NW, 2, N)
    hist_t = jnp.transpose(hist, (1, 2, 0))

    block_n = 1000 if N % 1000 == 0 else 8
    h = _prep(x, hist_t, a2, b2, block_n)

    epw_pad = -(-(E // NW) // (8 * K)) * (8 * K)
    E_pad = epw_pad * NW
    pad = E_pad - E
    src_p = jnp.concatenate([src, jnp.zeros((pad,), jnp.int32)])
    dst_p = jnp.concatenate([dst, jnp.full((pad,), N, jnp.int32)])
    dst2 = dst_p.reshape(E_pad // K, K)
    zeros = jnp.zeros((N + 8, D), jnp.float32)
    parts = _make_aggregate(E_pad, N, D)(h, src_p, dst2, zeros)

    return _finish(parts, hist_t, x, W, b, block_n)
